# baseline (speedup 1.0000x reference)
import functools

import jax
import jax.numpy as jnp
from jax.experimental import pallas as pl
from jax.experimental.pallas import tpu as pltpu
from jax.sharding import PartitionSpec as P

_Q, _K, _D = 1024, 100000, 128
_TEMP = 0.1
_QB = 1024
_KB = 2000
_NLOG2E_T = -1.4426950408889634 / _TEMP


def _body(le_ref, tl_ref, se_ref, swd_ref):
    kj = pl.program_id(0)

    @pl.when(kj == 0)
    def _init():
        se_ref[...] = jnp.zeros((_QB, 1), jnp.float32)
        swd_ref[...] = jnp.zeros((_QB, 1), jnp.float32)

    le = le_ref[...]
    lq = jnp.sum(le * le, axis=1, keepdims=True)
    le_n = le * jax.lax.rsqrt(jnp.maximum(lq, 1e-24))
    q2p = jnp.sum(le_n * le_n, axis=1, keepdims=True) + 1.0

    tl = tl_ref[...]
    tq = jnp.sum(tl * tl, axis=1, keepdims=True)
    tl_n = tl * jax.lax.rsqrt(jnp.maximum(tq, 1e-24))

    dot = jax.lax.dot_general(
        le_n.astype(jnp.bfloat16), tl_n.astype(jnp.bfloat16),
        (((1,), (1,)), ((), ())),
        preferred_element_type=jnp.float32)
    d2 = jnp.maximum(q2p - 2.0 * dot, 1e-12)
    d = d2 * jax.lax.rsqrt(d2)
    p = jnp.exp2(d * _NLOG2E_T)
    se_ref[...] += jnp.sum(p, axis=1, keepdims=True)
    swd_ref[...] += jnp.sum(p * d, axis=1, keepdims=True)


def _partial_sums(latent_eval, tl_shard):
    k_loc = tl_shard.shape[0]
    se, swd = pl.pallas_call(
        _body,
        grid=(k_loc // _KB,),
        in_specs=[
            pl.BlockSpec((_QB, _D), lambda kj: (0, 0)),
            pl.BlockSpec((_KB, _D), lambda kj: (kj, 0)),
        ],
        out_specs=[
            pl.BlockSpec((_QB, 1), lambda kj: (0, 0)),
            pl.BlockSpec((_QB, 1), lambda kj: (0, 0)),
        ],
        out_shape=[
            jax.ShapeDtypeStruct((_Q, 1), jnp.float32),
            jax.ShapeDtypeStruct((_Q, 1), jnp.float32),
        ],
        compiler_params=pltpu.CompilerParams(
            dimension_semantics=("arbitrary",),
        ),
    )(latent_eval, tl_shard)
    return se, swd


def kernel(latent_eval, train_latents):
    n_dev = len(jax.devices())
    if _K % (n_dev * _KB) != 0:
        n_dev = 1
    if n_dev == 1:
        se, swd = _partial_sums(latent_eval, train_latents)
        return (swd / se).reshape(_Q)

    mesh = jax.make_mesh((n_dev,), ("x",))

    @functools.partial(
        jax.shard_map, mesh=mesh,
        in_specs=(P(), P("x", None)), out_specs=P(), check_vma=False)
    def _sharded(le, tl_shard):
        se, swd = _partial_sums(le, tl_shard)
        se = jax.lax.psum(se, "x")
        swd = jax.lax.psum(swd, "x")
        return (swd / se).reshape(_Q)

    le_r = jax.reshard(latent_eval, jax.NamedSharding(mesh, P()))
    tl_r = jax.reshard(train_latents, jax.NamedSharding(mesh, P("x", None)))
    return _sharded(le_r, tl_r)


# speedup vs baseline: 2.3612x; 2.3612x over previous
import jax
import jax.numpy as jnp
from jax.experimental import pallas as pl
from jax.experimental.pallas import tpu as pltpu

_Q, _K, _D = 1024, 100000, 128
_TEMP = 0.1
_QB = 1024
_KB = 2048
_NSTEP = (_K + _KB - 1) // _KB
_NLOG2E_T = -1.4426950408889634 / _TEMP


def _body(le_ref, tl_ref, se_ref, swd_ref):
    kj = pl.program_id(0)
    nk = pl.num_programs(0)

    @pl.when(kj == 0)
    def _init():
        se_ref[...] = jnp.zeros((_QB, 1), jnp.float32)
        swd_ref[...] = jnp.zeros((_QB, 1), jnp.float32)

    le = le_ref[...]
    lq = jnp.sum(le * le, axis=1, keepdims=True)
    le_n = le * jax.lax.rsqrt(jnp.maximum(lq, 1e-24))
    q2p = jnp.sum(le_n * le_n, axis=1, keepdims=True) + 1.0

    tl = tl_ref[...]
    tq = jnp.sum(tl * tl, axis=1, keepdims=True)
    tl_n = tl * jax.lax.rsqrt(jnp.maximum(tq, 1e-24))

    dot = jax.lax.dot_general(
        le_n.astype(jnp.bfloat16), tl_n.astype(jnp.bfloat16),
        (((1,), (1,)), ((), ())),
        preferred_element_type=jnp.float32)
    d2 = jnp.maximum(q2p - 2.0 * dot, 1e-12)
    d = d2 * jax.lax.rsqrt(d2)
    p = jnp.exp2(d * _NLOG2E_T)

    @pl.when(kj < nk - 1)
    def _acc():
        se_ref[...] += jnp.sum(p, axis=1, keepdims=True)
        swd_ref[...] += jnp.sum(p * d, axis=1, keepdims=True)

    @pl.when(kj == nk - 1)
    def _acc_tail():
        lanes = jax.lax.broadcasted_iota(jnp.int32, (_QB, _KB), 1)
        valid = lanes < _K - (nk - 1) * _KB
        pm = jnp.where(valid, p, 0.0)
        pdm = jnp.where(valid, p * d, 0.0)
        se_ref[...] += jnp.sum(pm, axis=1, keepdims=True)
        swd_ref[...] += jnp.sum(pdm, axis=1, keepdims=True)


def kernel(latent_eval, train_latents):
    se, swd = pl.pallas_call(
        _body,
        grid=(_NSTEP,),
        in_specs=[
            pl.BlockSpec((_QB, _D), lambda kj: (0, 0)),
            pl.BlockSpec((_KB, _D), lambda kj: (kj, 0)),
        ],
        out_specs=[
            pl.BlockSpec((_QB, 1), lambda kj: (0, 0)),
            pl.BlockSpec((_QB, 1), lambda kj: (0, 0)),
        ],
        out_shape=[
            jax.ShapeDtypeStruct((_Q, 1), jnp.float32),
            jax.ShapeDtypeStruct((_Q, 1), jnp.float32),
        ],
        compiler_params=pltpu.CompilerParams(
            dimension_semantics=("arbitrary",),
        ),
    )(latent_eval, train_latents)
    return (swd / se).reshape(_Q)


# speedup vs baseline: 2.6776x; 1.1340x over previous
import jax
import jax.numpy as jnp
from jax.experimental import pallas as pl
from jax.experimental.pallas import tpu as pltpu

_Q, _K, _D = 1024, 100000, 128
_TEMP = 0.1
_KB = 2000
_NLOG2E_T = -1.4426950408889634 / _TEMP


def _body(le_ref, tl_ref, se_ref, swd_ref, les_ref, q2p_ref):
    kj = pl.program_id(0)

    @pl.when(kj == 0)
    def _init():
        le = le_ref[...]
        lq = jnp.sum(le * le, axis=1, keepdims=True)
        le_n = le * jax.lax.rsqrt(jnp.maximum(lq, 1e-24))
        q2p_ref[...] = jnp.sum(le_n * le_n, axis=1, keepdims=True) + 1.0
        les_ref[...] = (-2.0 * le_n).astype(jnp.bfloat16)
        se_ref[...] = jnp.zeros((_Q, 1), jnp.float32)
        swd_ref[...] = jnp.zeros((_Q, 1), jnp.float32)

    tl = tl_ref[...]
    tq = jnp.sum(tl * tl, axis=1, keepdims=True)
    tl_n = tl * jax.lax.rsqrt(jnp.maximum(tq, 1e-24))

    ndot = jax.lax.dot_general(
        les_ref[...], tl_n.astype(jnp.bfloat16),
        (((1,), (1,)), ((), ())),
        preferred_element_type=jnp.float32)
    d2 = jnp.maximum(q2p_ref[...] + ndot, 1e-12)
    d = d2 * jax.lax.rsqrt(d2)
    p = jnp.exp2(d * _NLOG2E_T)
    se_ref[...] += jnp.sum(p, axis=1, keepdims=True)
    swd_ref[...] += jnp.sum(p * d, axis=1, keepdims=True)


def kernel(latent_eval, train_latents):
    se, swd = pl.pallas_call(
        _body,
        grid=(_K // _KB,),
        in_specs=[
            pl.BlockSpec((_Q, _D), lambda kj: (0, 0)),
            pl.BlockSpec((_KB, _D), lambda kj: (kj, 0)),
        ],
        out_specs=[
            pl.BlockSpec((_Q, 1), lambda kj: (0, 0)),
            pl.BlockSpec((_Q, 1), lambda kj: (0, 0)),
        ],
        out_shape=[
            jax.ShapeDtypeStruct((_Q, 1), jnp.float32),
            jax.ShapeDtypeStruct((_Q, 1), jnp.float32),
        ],
        scratch_shapes=[
            pltpu.VMEM((_Q, _D), jnp.bfloat16),
            pltpu.VMEM((_Q, 1), jnp.float32),
        ],
        compiler_params=pltpu.CompilerParams(
            dimension_semantics=("arbitrary",),
        ),
    )(latent_eval, train_latents)
    return (swd / se).reshape(_Q)


# speedup vs baseline: 2.7220x; 1.0166x over previous
import jax
import jax.numpy as jnp
from jax.experimental import pallas as pl
from jax.experimental.pallas import tpu as pltpu

_Q, _K, _D = 1024, 100000, 128
_TEMP = 0.1
_KB = 4000
_NLOG2E_T = -1.4426950408889634 / _TEMP


def _body(le_ref, tl_ref, se_ref, swd_ref, les_ref, q2p_ref):
    kj = pl.program_id(0)

    @pl.when(kj == 0)
    def _init():
        le = le_ref[...]
        lq = jnp.sum(le * le, axis=1, keepdims=True)
        le_n = le * jax.lax.rsqrt(jnp.maximum(lq, 1e-24))
        q2p_ref[...] = jnp.sum(le_n * le_n, axis=1, keepdims=True) + 1.0
        les_ref[...] = (-2.0 * le_n).astype(jnp.bfloat16)
        se_ref[...] = jnp.zeros((_Q, 1), jnp.float32)
        swd_ref[...] = jnp.zeros((_Q, 1), jnp.float32)

    tl = tl_ref[...]
    tq = jnp.sum(tl * tl, axis=1, keepdims=True)
    tl_n = tl * jax.lax.rsqrt(jnp.maximum(tq, 1e-24))

    ndot = jax.lax.dot_general(
        les_ref[...], tl_n.astype(jnp.bfloat16),
        (((1,), (1,)), ((), ())),
        preferred_element_type=jnp.float32)
    d2 = jnp.maximum(q2p_ref[...] + ndot, 1e-12)
    d = d2 * jax.lax.rsqrt(d2)
    p = jnp.exp2(d * _NLOG2E_T)
    se_ref[...] += jnp.sum(p, axis=1, keepdims=True)
    swd_ref[...] += jnp.sum(p * d, axis=1, keepdims=True)


def kernel(latent_eval, train_latents):
    se, swd = pl.pallas_call(
        _body,
        grid=(_K // _KB,),
        in_specs=[
            pl.BlockSpec((_Q, _D), lambda kj: (0, 0)),
            pl.BlockSpec((_KB, _D), lambda kj: (kj, 0)),
        ],
        out_specs=[
            pl.BlockSpec((_Q, 1), lambda kj: (0, 0)),
            pl.BlockSpec((_Q, 1), lambda kj: (0, 0)),
        ],
        out_shape=[
            jax.ShapeDtypeStruct((_Q, 1), jnp.float32),
            jax.ShapeDtypeStruct((_Q, 1), jnp.float32),
        ],
        scratch_shapes=[
            pltpu.VMEM((_Q, _D), jnp.bfloat16),
            pltpu.VMEM((_Q, 1), jnp.float32),
        ],
        compiler_params=pltpu.CompilerParams(
            dimension_semantics=("arbitrary",),
        ),
    )(latent_eval, train_latents)
    return (swd / se).reshape(_Q)


# speedup vs baseline: 2.7613x; 1.0144x over previous
import jax
import jax.numpy as jnp
from jax.experimental import pallas as pl
from jax.experimental.pallas import tpu as pltpu

_Q, _K, _D = 1024, 100000, 128
_TEMP = 0.1
_KB = 5000
_NLOG2E_T = -1.4426950408889634 / _TEMP


def _body(le_ref, tl_ref, se_ref, swd_ref, les_ref, q2p_ref):
    kj = pl.program_id(0)

    @pl.when(kj == 0)
    def _init():
        le = le_ref[...]
        lq = jnp.sum(le * le, axis=1, keepdims=True)
        le_n = le * jax.lax.rsqrt(jnp.maximum(lq, 1e-24))
        q2p_ref[...] = jnp.sum(le_n * le_n, axis=1, keepdims=True) + 1.0
        les_ref[...] = (-2.0 * le_n).astype(jnp.bfloat16)
        se_ref[...] = jnp.zeros((_Q, 1), jnp.float32)
        swd_ref[...] = jnp.zeros((_Q, 1), jnp.float32)

    tl = tl_ref[...]
    tq = jnp.sum(tl * tl, axis=1, keepdims=True)
    tl_n = tl * jax.lax.rsqrt(jnp.maximum(tq, 1e-24))

    ndot = jax.lax.dot_general(
        les_ref[...], tl_n.astype(jnp.bfloat16),
        (((1,), (1,)), ((), ())),
        preferred_element_type=jnp.float32)
    d2 = jnp.maximum(q2p_ref[...] + ndot, 1e-12)
    d = d2 * jax.lax.rsqrt(d2)
    p = jnp.exp2(d * _NLOG2E_T)
    se_ref[...] += jnp.sum(p, axis=1, keepdims=True)
    swd_ref[...] += jnp.sum(p * d, axis=1, keepdims=True)


def kernel(latent_eval, train_latents):
    se, swd = pl.pallas_call(
        _body,
        grid=(_K // _KB,),
        in_specs=[
            pl.BlockSpec((_Q, _D), lambda kj: (0, 0)),
            pl.BlockSpec((_KB, _D), lambda kj: (kj, 0)),
        ],
        out_specs=[
            pl.BlockSpec((_Q, 1), lambda kj: (0, 0)),
            pl.BlockSpec((_Q, 1), lambda kj: (0, 0)),
        ],
        out_shape=[
            jax.ShapeDtypeStruct((_Q, 1), jnp.float32),
            jax.ShapeDtypeStruct((_Q, 1), jnp.float32),
        ],
        scratch_shapes=[
            pltpu.VMEM((_Q, _D), jnp.bfloat16),
            pltpu.VMEM((_Q, 1), jnp.float32),
        ],
        compiler_params=pltpu.CompilerParams(
            dimension_semantics=("arbitrary",),
        ),
    )(latent_eval, train_latents)
    return (swd / se).reshape(_Q)


# speedup vs baseline: 2.9576x; 1.0711x over previous
import jax
import jax.numpy as jnp
from jax.experimental import pallas as pl
from jax.experimental.pallas import tpu as pltpu

_Q, _K, _D = 1024, 100000, 128
_TEMP = 0.1
_KB = 10000
_NLOG2E_T = -1.4426950408889634 / _TEMP


def _body(le_ref, tl_ref, se_ref, swd_ref, les_ref, q2p_ref):
    kj = pl.program_id(0)

    @pl.when(kj == 0)
    def _init():
        le = le_ref[...]
        lq = jnp.sum(le * le, axis=1, keepdims=True)
        le_n = le * jax.lax.rsqrt(jnp.maximum(lq, 1e-24))
        q2p_ref[...] = jnp.sum(le_n * le_n, axis=1, keepdims=True) + 1.0
        les_ref[...] = (-2.0 * le_n).astype(jnp.bfloat16)
        se_ref[...] = jnp.zeros((_Q, 1), jnp.float32)
        swd_ref[...] = jnp.zeros((_Q, 1), jnp.float32)

    tl = tl_ref[...]
    tq = jnp.sum(tl * tl, axis=1, keepdims=True)
    tl_n = tl * jax.lax.rsqrt(jnp.maximum(tq, 1e-24))

    ndot = jax.lax.dot_general(
        les_ref[...], tl_n.astype(jnp.bfloat16),
        (((1,), (1,)), ((), ())),
        preferred_element_type=jnp.float32)
    d2 = jnp.maximum(q2p_ref[...] + ndot, 1e-12)
    d = d2 * jax.lax.rsqrt(d2)
    p = jnp.exp2(d * _NLOG2E_T)
    se_ref[...] += jnp.sum(p, axis=1, keepdims=True)
    swd_ref[...] += jnp.sum(p * d, axis=1, keepdims=True)


def kernel(latent_eval, train_latents):
    se, swd = pl.pallas_call(
        _body,
        grid=(_K // _KB,),
        in_specs=[
            pl.BlockSpec((_Q, _D), lambda kj: (0, 0)),
            pl.BlockSpec((_KB, _D), lambda kj: (kj, 0)),
        ],
        out_specs=[
            pl.BlockSpec((_Q, 1), lambda kj: (0, 0)),
            pl.BlockSpec((_Q, 1), lambda kj: (0, 0)),
        ],
        out_shape=[
            jax.ShapeDtypeStruct((_Q, 1), jnp.float32),
            jax.ShapeDtypeStruct((_Q, 1), jnp.float32),
        ],
        scratch_shapes=[
            pltpu.VMEM((_Q, _D), jnp.bfloat16),
            pltpu.VMEM((_Q, 1), jnp.float32),
        ],
        compiler_params=pltpu.CompilerParams(
            dimension_semantics=("arbitrary",),
        ),
    )(latent_eval, train_latents)
    return (swd / se).reshape(_Q)
